# baseline (speedup 1.0000x reference)
import functools

import jax
import jax.numpy as jnp
from jax import lax
from jax.experimental import pallas as pl
from jax.experimental.pallas import tpu as pltpu
from jax.experimental.pallas import tpu_sc as plsc

_N = 10000
_E = 320000
_D = 128
_R = 8
_NB = 4

_BN = 1000
_GRID = _N // _BN
_EB = 128
_NBATCH = _E // _EB
_NWORK = 32
_T = 80
_NBATCH_PAD = _T * _NWORK
_NDUMP = 512
_NP = _N + _NDUMP
_RPS = 624
_RTAIL = _N - 16 * _RPS


def _gidx_body(et_ref, src_ref, out_ref):
    out_ref[...] = et_ref[...] * _N + src_ref[...]


def _make_gidx(etype, src):
    et2 = etype.reshape(_NBATCH, _EB)
    s2 = src.reshape(_NBATCH, _EB)
    return pl.pallas_call(
        _gidx_body,
        out_shape=jax.ShapeDtypeStruct((_NBATCH, _EB), jnp.int32),
    )(et2, s2)


def _transform_body(comp_ref, h_ref, V_ref, Wself_ref, b_ref, ft_ref, self_ref):
    h = h_ref[...]
    hb = [jnp.dot(h, V_ref[bb], preferred_element_type=jnp.float32)
          for bb in range(_NB)]
    for r in range(_R):
        acc = comp_ref[r, 0] * hb[0]
        for bb in range(1, _NB):
            acc = acc + comp_ref[r, bb] * hb[bb]
        ft_ref[r] = acc
    self_ref[...] = (jnp.dot(h, Wself_ref[...], preferred_element_type=jnp.float32)
                     + b_ref[...])


def _transform(h, V, comp, Wself, b):
    return pl.pallas_call(
        _transform_body,
        grid=(_GRID,),
        in_specs=[
            pl.BlockSpec(memory_space=pltpu.SMEM),
            pl.BlockSpec((_BN, _D), lambda i: (i, 0)),
            pl.BlockSpec((_NB, _D, _D), lambda i: (0, 0, 0)),
            pl.BlockSpec((_D, _D), lambda i: (0, 0)),
            pl.BlockSpec((1, _D), lambda i: (0, 0)),
        ],
        out_specs=[
            pl.BlockSpec((_R, _BN, _D), lambda i: (0, i, 0)),
            pl.BlockSpec((_BN, _D), lambda i: (i, 0)),
        ],
        out_shape=[
            jax.ShapeDtypeStruct((_R, _N, _D), jnp.float32),
            jax.ShapeDtypeStruct((_N, _D), jnp.float32),
        ],
    )(comp, h, V, Wself, b.reshape(1, _D))


def _transform_fused_body(comp_ref, agg_ref, selfp_ref, hprev_ref, V_ref,
                          Wself_ref, b_ref, h_ref, ft_ref, self_ref):
    h = (jnp.maximum(agg_ref[0] + agg_ref[1] + selfp_ref[...], 0.0)
         + hprev_ref[...])
    h_ref[...] = h
    hb = [jnp.dot(h, V_ref[bb], preferred_element_type=jnp.float32)
          for bb in range(_NB)]
    for r in range(_R):
        acc = comp_ref[r, 0] * hb[0]
        for bb in range(1, _NB):
            acc = acc + comp_ref[r, bb] * hb[bb]
        ft_ref[r] = acc
    self_ref[...] = (jnp.dot(h, Wself_ref[...], preferred_element_type=jnp.float32)
                     + b_ref[...])


def _transform_fused(agg, selfp, hprev, V, comp, Wself, b):
    return pl.pallas_call(
        _transform_fused_body,
        grid=(_GRID,),
        in_specs=[
            pl.BlockSpec(memory_space=pltpu.SMEM),
            pl.BlockSpec((2, _BN, _D), lambda i: (0, i, 0)),
            pl.BlockSpec((_BN, _D), lambda i: (i, 0)),
            pl.BlockSpec((_BN, _D), lambda i: (i, 0)),
            pl.BlockSpec((_NB, _D, _D), lambda i: (0, 0, 0)),
            pl.BlockSpec((_D, _D), lambda i: (0, 0)),
            pl.BlockSpec((1, _D), lambda i: (0, 0)),
        ],
        out_specs=[
            pl.BlockSpec((_BN, _D), lambda i: (i, 0)),
            pl.BlockSpec((_R, _BN, _D), lambda i: (0, i, 0)),
            pl.BlockSpec((_BN, _D), lambda i: (i, 0)),
        ],
        out_shape=[
            jax.ShapeDtypeStruct((_N, _D), jnp.float32),
            jax.ShapeDtypeStruct((_R, _N, _D), jnp.float32),
            jax.ShapeDtypeStruct((_N, _D), jnp.float32),
        ],
    )(comp, agg, selfp, hprev, V, Wself, b.reshape(1, _D))


def _epilogue_body(agg_ref, selfp_ref, hprev_ref, out_ref):
    out_ref[...] = (jnp.maximum(agg_ref[0] + agg_ref[1] + selfp_ref[...], 0.0)
                    + hprev_ref[...])


def _epilogue(agg, selfp, hprev):
    return pl.pallas_call(
        _epilogue_body,
        grid=(_GRID,),
        in_specs=[
            pl.BlockSpec((2, _BN, _D), lambda i: (0, i, 0)),
            pl.BlockSpec((_BN, _D), lambda i: (i, 0)),
            pl.BlockSpec((_BN, _D), lambda i: (i, 0)),
        ],
        out_specs=pl.BlockSpec((_BN, _D), lambda i: (i, 0)),
        out_shape=jax.ShapeDtypeStruct((_N, _D), jnp.float32),
    )(agg, selfp, hprev)


def _sc_agg_body(table_h, idx_h, zer_h, out_h,
                 pck0, pck1, rows0, rows1,
                 acc_s, semg0, semg1, semi):
    cid = lax.axis_index("c")
    sid = lax.axis_index("s")
    wid = sid * 2 + cid

    r0 = sid * _RPS
    pltpu.sync_copy(zer_h.at[pl.ds(r0, _RPS)], acc_s.at[pl.ds(r0, _RPS)])

    @pl.when(sid == 15)
    def _zero_tail():
        pltpu.sync_copy(zer_h.at[pl.ds(0, _RTAIL)],
                        acc_s.at[pl.ds(16 * _RPS, _RTAIL)])

    plsc.subcore_barrier()

    pck = (pck0, pck1)
    rows = (rows0, rows1)
    semg = (semg0, semg1)

    for i in range(_T):
        k = i % 2
        pltpu.sync_copy(idx_h.at[wid + i * _NWORK], pck[k])
        pltpu.async_copy(table_h.at[pck[k].at[0]], rows[k], semg[k]).wait()
        pltpu.sync_copy(rows[k], acc_s.at[pck[k].at[1]], add=True)

    plsc.subcore_barrier()

    pltpu.sync_copy(acc_s.at[pl.ds(r0, _RPS)], out_h.at[cid, pl.ds(r0, _RPS)])

    @pl.when(sid == 15)
    def _write_tail():
        pltpu.sync_copy(acc_s.at[pl.ds(16 * _RPS, _RTAIL)],
                        out_h.at[cid, pl.ds(16 * _RPS, _RTAIL)])


@functools.cache
def _sc_agg_kernel():
    mesh = plsc.VectorSubcoreMesh(
        core_axis_name="c", subcore_axis_name="s", num_cores=2, num_subcores=16)
    return pl.kernel(
        _sc_agg_body,
        out_type=jax.ShapeDtypeStruct((2, _N, _D), jnp.float32),
        mesh=mesh,
        scratch_types=[
            pltpu.VMEM((2, _EB), jnp.int32),
            pltpu.VMEM((2, _EB), jnp.int32),
            pltpu.VMEM((_EB, _D), jnp.float32),
            pltpu.VMEM((_EB, _D), jnp.float32),
            pltpu.VMEM_SHARED((_NP, _D), jnp.float32),
            pltpu.SemaphoreType.DMA,
            pltpu.SemaphoreType.DMA,
            pltpu.SemaphoreType.DMA,
        ],
    )


def _sc_agg(table, idx_packed, zer):
    return _sc_agg_kernel()(table, idx_packed, zer)


def kernel(feat, g, etype, V1, comp1, Wself1, b1, V2, comp2, Wself2, b2,
           V3, comp3, Wself3, b3):
    src = g[0]
    dst = g[1]
    npad = _NBATCH_PAD - _NBATCH
    gidx2d = jnp.concatenate(
        [_make_gidx(etype, src), jnp.zeros((npad, _EB), jnp.int32)])
    dump = (_N + (jnp.arange(npad * _EB, dtype=jnp.int32) % _NDUMP)
            ).reshape(npad, _EB)
    dst2d = jnp.concatenate([dst.reshape(_NBATCH, _EB), dump])
    idx_packed = jnp.stack([gidx2d, dst2d], axis=1)
    zer = jnp.zeros((_N, _D), jnp.float32)

    ft1, self1 = _transform(feat, V1, comp1, Wself1, b1)
    agg1 = _sc_agg(ft1.reshape(_R * _N, _D), idx_packed, zer)
    h1, ft2, self2 = _transform_fused(agg1, self1, feat, V2, comp2, Wself2, b2)
    agg2 = _sc_agg(ft2.reshape(_R * _N, _D), idx_packed, zer)
    h2, ft3, self3 = _transform_fused(agg2, self2, h1, V3, comp3, Wself3, b3)
    agg3 = _sc_agg(ft3.reshape(_R * _N, _D), idx_packed, zer)
    return _epilogue(agg3, self3, h2)


# speedup vs baseline: 3.2021x; 3.2021x over previous
import functools

import jax
import jax.numpy as jnp
from jax import lax
from jax.experimental import pallas as pl
from jax.experimental.pallas import tpu as pltpu
from jax.experimental.pallas import tpu_sc as plsc

_N = 10000
_E = 320000
_D = 128
_R = 8
_NB = 4

_BN = 1000
_GRID = _N // _BN
_EB = 128
_NBATCH = _E // _EB
_NWORK = 32
_T = 80
_NBATCH_PAD = _T * _NWORK
_NDUMP = 512
_NP = _N + _NDUMP
_RPS = 624
_RTAIL = _N - 16 * _RPS


def _gidx_body(et_ref, src_ref, out_ref):
    out_ref[...] = et_ref[...] * _N + src_ref[...]


def _make_gidx(etype, src):
    et2 = etype.reshape(_NBATCH, _EB)
    s2 = src.reshape(_NBATCH, _EB)
    return pl.pallas_call(
        _gidx_body,
        out_shape=jax.ShapeDtypeStruct((_NBATCH, _EB), jnp.int32),
    )(et2, s2)


def _transform_body(comp_ref, h_ref, V_ref, Wself_ref, b_ref, ft_ref, self_ref):
    h = h_ref[...]
    hb = [jnp.dot(h, V_ref[bb], preferred_element_type=jnp.float32)
          for bb in range(_NB)]
    for r in range(_R):
        acc = comp_ref[r, 0] * hb[0]
        for bb in range(1, _NB):
            acc = acc + comp_ref[r, bb] * hb[bb]
        ft_ref[r] = acc
    self_ref[...] = (jnp.dot(h, Wself_ref[...], preferred_element_type=jnp.float32)
                     + b_ref[...])


def _transform(h, V, comp, Wself, b):
    return pl.pallas_call(
        _transform_body,
        grid=(_GRID,),
        in_specs=[
            pl.BlockSpec(memory_space=pltpu.SMEM),
            pl.BlockSpec((_BN, _D), lambda i: (i, 0)),
            pl.BlockSpec((_NB, _D, _D), lambda i: (0, 0, 0)),
            pl.BlockSpec((_D, _D), lambda i: (0, 0)),
            pl.BlockSpec((1, _D), lambda i: (0, 0)),
        ],
        out_specs=[
            pl.BlockSpec((_R, _BN, _D), lambda i: (0, i, 0)),
            pl.BlockSpec((_BN, _D), lambda i: (i, 0)),
        ],
        out_shape=[
            jax.ShapeDtypeStruct((_R, _N, _D), jnp.float32),
            jax.ShapeDtypeStruct((_N, _D), jnp.float32),
        ],
    )(comp, h, V, Wself, b.reshape(1, _D))


def _transform_fused_body(comp_ref, agg_ref, selfp_ref, hprev_ref, V_ref,
                          Wself_ref, b_ref, h_ref, ft_ref, self_ref):
    h = (jnp.maximum(agg_ref[0] + agg_ref[1] + selfp_ref[...], 0.0)
         + hprev_ref[...])
    h_ref[...] = h
    hb = [jnp.dot(h, V_ref[bb], preferred_element_type=jnp.float32)
          for bb in range(_NB)]
    for r in range(_R):
        acc = comp_ref[r, 0] * hb[0]
        for bb in range(1, _NB):
            acc = acc + comp_ref[r, bb] * hb[bb]
        ft_ref[r] = acc
    self_ref[...] = (jnp.dot(h, Wself_ref[...], preferred_element_type=jnp.float32)
                     + b_ref[...])


def _transform_fused(agg, selfp, hprev, V, comp, Wself, b):
    return pl.pallas_call(
        _transform_fused_body,
        grid=(_GRID,),
        in_specs=[
            pl.BlockSpec(memory_space=pltpu.SMEM),
            pl.BlockSpec((2, _BN, _D), lambda i: (0, i, 0)),
            pl.BlockSpec((_BN, _D), lambda i: (i, 0)),
            pl.BlockSpec((_BN, _D), lambda i: (i, 0)),
            pl.BlockSpec((_NB, _D, _D), lambda i: (0, 0, 0)),
            pl.BlockSpec((_D, _D), lambda i: (0, 0)),
            pl.BlockSpec((1, _D), lambda i: (0, 0)),
        ],
        out_specs=[
            pl.BlockSpec((_BN, _D), lambda i: (i, 0)),
            pl.BlockSpec((_R, _BN, _D), lambda i: (0, i, 0)),
            pl.BlockSpec((_BN, _D), lambda i: (i, 0)),
        ],
        out_shape=[
            jax.ShapeDtypeStruct((_N, _D), jnp.float32),
            jax.ShapeDtypeStruct((_R, _N, _D), jnp.float32),
            jax.ShapeDtypeStruct((_N, _D), jnp.float32),
        ],
    )(comp, agg, selfp, hprev, V, Wself, b.reshape(1, _D))


def _epilogue_body(agg_ref, selfp_ref, hprev_ref, out_ref):
    out_ref[...] = (jnp.maximum(agg_ref[0] + agg_ref[1] + selfp_ref[...], 0.0)
                    + hprev_ref[...])


def _epilogue(agg, selfp, hprev):
    return pl.pallas_call(
        _epilogue_body,
        grid=(_GRID,),
        in_specs=[
            pl.BlockSpec((2, _BN, _D), lambda i: (0, i, 0)),
            pl.BlockSpec((_BN, _D), lambda i: (i, 0)),
            pl.BlockSpec((_BN, _D), lambda i: (i, 0)),
        ],
        out_specs=pl.BlockSpec((_BN, _D), lambda i: (i, 0)),
        out_shape=jax.ShapeDtypeStruct((_N, _D), jnp.float32),
    )(agg, selfp, hprev)


def _sc_agg_body(table_h, idx_h, zer_h, out_h,
                 pck0, pck1, rows0, rows1,
                 acc_s, semg0, semg1, semi):
    cid = lax.axis_index("c")
    sid = lax.axis_index("s")
    wid = sid * 2 + cid

    r0 = sid * _RPS
    pltpu.sync_copy(zer_h.at[pl.ds(r0, _RPS)], acc_s.at[pl.ds(r0, _RPS)])

    @pl.when(sid == 15)
    def _zero_tail():
        pltpu.sync_copy(zer_h.at[pl.ds(0, _RTAIL)],
                        acc_s.at[pl.ds(16 * _RPS, _RTAIL)])

    plsc.subcore_barrier()

    pck = (pck0, pck1)
    rows = (rows0, rows1)
    semg = (semg0, semg1)

    pltpu.sync_copy(idx_h.at[wid], pck[0])
    gd = pltpu.async_copy(table_h.at[pck[0].at[0]], rows[0], semg[0])
    idxd = pltpu.async_copy(idx_h.at[wid + _NWORK], pck[1], semi)
    gdesc = [gd, None]

    for i in range(_T):
        k = i % 2
        k1 = 1 - k
        if i + 1 < _T:
            idxd.wait()
            gdesc[k1] = pltpu.async_copy(table_h.at[pck[k1].at[0]],
                                         rows[k1], semg[k1])
        gdesc[k].wait()
        pltpu.sync_copy(rows[k], acc_s.at[pck[k].at[1]], add=True)
        if i + 2 < _T:
            idxd = pltpu.async_copy(idx_h.at[wid + (i + 2) * _NWORK],
                                    pck[k], semi)

    plsc.subcore_barrier()

    pltpu.sync_copy(acc_s.at[pl.ds(r0, _RPS)], out_h.at[cid, pl.ds(r0, _RPS)])

    @pl.when(sid == 15)
    def _write_tail():
        pltpu.sync_copy(acc_s.at[pl.ds(16 * _RPS, _RTAIL)],
                        out_h.at[cid, pl.ds(16 * _RPS, _RTAIL)])


@functools.cache
def _sc_agg_kernel():
    mesh = plsc.VectorSubcoreMesh(
        core_axis_name="c", subcore_axis_name="s", num_cores=2, num_subcores=16)
    return pl.kernel(
        _sc_agg_body,
        out_type=jax.ShapeDtypeStruct((2, _N, _D), jnp.float32),
        mesh=mesh,
        scratch_types=[
            pltpu.VMEM((2, _EB), jnp.int32),
            pltpu.VMEM((2, _EB), jnp.int32),
            pltpu.VMEM((_EB, _D), jnp.float32),
            pltpu.VMEM((_EB, _D), jnp.float32),
            pltpu.VMEM_SHARED((_NP, _D), jnp.float32),
            pltpu.SemaphoreType.DMA,
            pltpu.SemaphoreType.DMA,
            pltpu.SemaphoreType.DMA,
        ],
    )


def _sc_agg(table, idx_packed, zer):
    return _sc_agg_kernel()(table, idx_packed, zer)


def kernel(feat, g, etype, V1, comp1, Wself1, b1, V2, comp2, Wself2, b2,
           V3, comp3, Wself3, b3):
    src = g[0]
    dst = g[1]
    npad = _NBATCH_PAD - _NBATCH
    padg = (jnp.arange(npad * _EB, dtype=jnp.int32) * 61 % (_R * _N)
            ).reshape(npad, _EB)
    gidx2d = jnp.concatenate([_make_gidx(etype, src), padg])
    dump = (_N + (jnp.arange(npad * _EB, dtype=jnp.int32) % _NDUMP)
            ).reshape(npad, _EB)
    dst2d = jnp.concatenate([dst.reshape(_NBATCH, _EB), dump])
    idx_packed = jnp.stack([gidx2d, dst2d], axis=1)
    zer = jnp.zeros((_N, _D), jnp.float32)

    ft1, self1 = _transform(feat, V1, comp1, Wself1, b1)
    agg1 = _sc_agg(ft1.reshape(_R * _N, _D), idx_packed, zer)
    h1, ft2, self2 = _transform_fused(agg1, self1, feat, V2, comp2, Wself2, b2)
    agg2 = _sc_agg(ft2.reshape(_R * _N, _D), idx_packed, zer)
    h2, ft3, self3 = _transform_fused(agg2, self2, h1, V3, comp3, Wself3, b3)
    agg3 = _sc_agg(ft3.reshape(_R * _N, _D), idx_packed, zer)
    return _epilogue(agg3, self3, h2)


# speedup vs baseline: 3.5492x; 1.1084x over previous
import functools

import jax
import jax.numpy as jnp
from jax import lax
from jax.experimental import pallas as pl
from jax.experimental.pallas import tpu as pltpu
from jax.experimental.pallas import tpu_sc as plsc

_N = 10000
_E = 320000
_D = 128
_R = 8
_NB = 4

_BN = 1000
_GRID = _N // _BN
_EB = 128
_NBATCH = _E // _EB
_NWORK = 32
_T = 80
_NBATCH_PAD = _T * _NWORK
_NDUMP = 64
_NP = _N + _NDUMP
_RPS = 624
_RTAIL = _N - 16 * _RPS


def _gidx_body(et_ref, src_ref, out_ref):
    out_ref[...] = et_ref[...] * _N + src_ref[...]


def _make_gidx(etype, src):
    et2 = etype.reshape(_NBATCH, _EB)
    s2 = src.reshape(_NBATCH, _EB)
    return pl.pallas_call(
        _gidx_body,
        out_shape=jax.ShapeDtypeStruct((_NBATCH, _EB), jnp.int32),
    )(et2, s2)


def _transform_body(comp_ref, h_ref, V_ref, Wself_ref, b_ref, ft_ref, self_ref):
    h = h_ref[...]
    hb = [jnp.dot(h, V_ref[bb], preferred_element_type=jnp.float32)
          for bb in range(_NB)]
    for r in range(_R):
        acc = comp_ref[r, 0] * hb[0]
        for bb in range(1, _NB):
            acc = acc + comp_ref[r, bb] * hb[bb]
        ft_ref[r] = acc
    self_ref[...] = (jnp.dot(h, Wself_ref[...], preferred_element_type=jnp.float32)
                     + b_ref[...])


def _transform(h, V, comp, Wself, b):
    return pl.pallas_call(
        _transform_body,
        grid=(_GRID,),
        in_specs=[
            pl.BlockSpec(memory_space=pltpu.SMEM),
            pl.BlockSpec((_BN, _D), lambda i: (i, 0)),
            pl.BlockSpec((_NB, _D, _D), lambda i: (0, 0, 0)),
            pl.BlockSpec((_D, _D), lambda i: (0, 0)),
            pl.BlockSpec((1, _D), lambda i: (0, 0)),
        ],
        out_specs=[
            pl.BlockSpec((_R, _BN, _D), lambda i: (0, i, 0)),
            pl.BlockSpec((_BN, _D), lambda i: (i, 0)),
        ],
        out_shape=[
            jax.ShapeDtypeStruct((_R, _N, _D), jnp.float32),
            jax.ShapeDtypeStruct((_N, _D), jnp.float32),
        ],
    )(comp, h, V, Wself, b.reshape(1, _D))


def _transform_fused_body(comp_ref, agg_ref, selfp_ref, hprev_ref, V_ref,
                          Wself_ref, b_ref, h_ref, ft_ref, self_ref):
    h = (jnp.maximum(agg_ref[0] + agg_ref[1] + selfp_ref[...], 0.0)
         + hprev_ref[...])
    h_ref[...] = h
    hb = [jnp.dot(h, V_ref[bb], preferred_element_type=jnp.float32)
          for bb in range(_NB)]
    for r in range(_R):
        acc = comp_ref[r, 0] * hb[0]
        for bb in range(1, _NB):
            acc = acc + comp_ref[r, bb] * hb[bb]
        ft_ref[r] = acc
    self_ref[...] = (jnp.dot(h, Wself_ref[...], preferred_element_type=jnp.float32)
                     + b_ref[...])


def _transform_fused(agg, selfp, hprev, V, comp, Wself, b):
    return pl.pallas_call(
        _transform_fused_body,
        grid=(_GRID,),
        in_specs=[
            pl.BlockSpec(memory_space=pltpu.SMEM),
            pl.BlockSpec((2, _BN, _D), lambda i: (0, i, 0)),
            pl.BlockSpec((_BN, _D), lambda i: (i, 0)),
            pl.BlockSpec((_BN, _D), lambda i: (i, 0)),
            pl.BlockSpec((_NB, _D, _D), lambda i: (0, 0, 0)),
            pl.BlockSpec((_D, _D), lambda i: (0, 0)),
            pl.BlockSpec((1, _D), lambda i: (0, 0)),
        ],
        out_specs=[
            pl.BlockSpec((_BN, _D), lambda i: (i, 0)),
            pl.BlockSpec((_R, _BN, _D), lambda i: (0, i, 0)),
            pl.BlockSpec((_BN, _D), lambda i: (i, 0)),
        ],
        out_shape=[
            jax.ShapeDtypeStruct((_N, _D), jnp.float32),
            jax.ShapeDtypeStruct((_R, _N, _D), jnp.float32),
            jax.ShapeDtypeStruct((_N, _D), jnp.float32),
        ],
    )(comp, agg, selfp, hprev, V, Wself, b.reshape(1, _D))


def _epilogue_body(agg_ref, selfp_ref, hprev_ref, out_ref):
    out_ref[...] = (jnp.maximum(agg_ref[0] + agg_ref[1] + selfp_ref[...], 0.0)
                    + hprev_ref[...])


def _epilogue(agg, selfp, hprev):
    return pl.pallas_call(
        _epilogue_body,
        grid=(_GRID,),
        in_specs=[
            pl.BlockSpec((2, _BN, _D), lambda i: (0, i, 0)),
            pl.BlockSpec((_BN, _D), lambda i: (i, 0)),
            pl.BlockSpec((_BN, _D), lambda i: (i, 0)),
        ],
        out_specs=pl.BlockSpec((_BN, _D), lambda i: (i, 0)),
        out_shape=jax.ShapeDtypeStruct((_N, _D), jnp.float32),
    )(agg, selfp, hprev)


def _sc_agg_body(table_h, idx_h, zer_h, out_h,
                 pck0, pck1, pck2, rows0, rows1, rows2,
                 acc_s, semg0, semg1, semg2, sems0, sems1, sems2, semi):
    cid = lax.axis_index("c")
    sid = lax.axis_index("s")
    wid = sid * 2 + cid

    r0 = sid * _RPS
    pltpu.sync_copy(zer_h.at[pl.ds(r0, _RPS)], acc_s.at[pl.ds(r0, _RPS)])

    @pl.when(sid == 15)
    def _zero_tail():
        pltpu.sync_copy(zer_h.at[pl.ds(0, _RTAIL)],
                        acc_s.at[pl.ds(16 * _RPS, _RTAIL)])

    plsc.subcore_barrier()

    pck = (pck0, pck1, pck2)
    rows = (rows0, rows1, rows2)
    semg = (semg0, semg1, semg2)
    sems = (sems0, sems1, sems2)

    pltpu.sync_copy(idx_h.at[wid], pck[0])
    gdesc = [pltpu.async_copy(table_h.at[pck[0].at[0]], rows[0], semg[0]),
             None, None]
    sdesc = [None, None, None]
    idxd = pltpu.async_copy(idx_h.at[wid + _NWORK], pck[1], semi)

    for i in range(_T):
        k = i % 3
        kn = (i + 1) % 3
        if i + 1 < _T:
            idxd.wait()
            gdesc[kn] = pltpu.async_copy(table_h.at[pck[kn].at[0]],
                                         rows[kn], semg[kn])
        gdesc[k].wait()
        if i >= 1:
            sdesc[(i - 1) % 3].wait()
        sdesc[k] = pltpu.async_copy(rows[k], acc_s.at[pck[k].at[1]],
                                    sems[k], add=True)
        if i + 2 < _T:
            idxd = pltpu.async_copy(idx_h.at[wid + (i + 2) * _NWORK],
                                    pck[(i + 2) % 3], semi)

    sdesc[(_T - 1) % 3].wait()
    plsc.subcore_barrier()

    pltpu.sync_copy(acc_s.at[pl.ds(r0, _RPS)], out_h.at[cid, pl.ds(r0, _RPS)])

    @pl.when(sid == 15)
    def _write_tail():
        pltpu.sync_copy(acc_s.at[pl.ds(16 * _RPS, _RTAIL)],
                        out_h.at[cid, pl.ds(16 * _RPS, _RTAIL)])


@functools.cache
def _sc_agg_kernel():
    mesh = plsc.VectorSubcoreMesh(
        core_axis_name="c", subcore_axis_name="s", num_cores=2, num_subcores=16)
    return pl.kernel(
        _sc_agg_body,
        out_type=jax.ShapeDtypeStruct((2, _N, _D), jnp.float32),
        mesh=mesh,
        scratch_types=[
            pltpu.VMEM((2, _EB), jnp.int32),
            pltpu.VMEM((2, _EB), jnp.int32),
            pltpu.VMEM((2, _EB), jnp.int32),
            pltpu.VMEM((_EB, _D), jnp.float32),
            pltpu.VMEM((_EB, _D), jnp.float32),
            pltpu.VMEM((_EB, _D), jnp.float32),
            pltpu.VMEM_SHARED((_NP, _D), jnp.float32),
            pltpu.SemaphoreType.DMA,
            pltpu.SemaphoreType.DMA,
            pltpu.SemaphoreType.DMA,
            pltpu.SemaphoreType.DMA,
            pltpu.SemaphoreType.DMA,
            pltpu.SemaphoreType.DMA,
            pltpu.SemaphoreType.DMA,
        ],
    )


def _sc_agg(table, idx_packed, zer):
    return _sc_agg_kernel()(table, idx_packed, zer)


def kernel(feat, g, etype, V1, comp1, Wself1, b1, V2, comp2, Wself2, b2,
           V3, comp3, Wself3, b3):
    src = g[0]
    dst = g[1]
    npad = _NBATCH_PAD - _NBATCH
    padg = (jnp.arange(npad * _EB, dtype=jnp.int32) * 61 % (_R * _N)
            ).reshape(npad, _EB)
    gidx2d = jnp.concatenate([_make_gidx(etype, src), padg])
    dump = (_N + (jnp.arange(npad * _EB, dtype=jnp.int32) % _NDUMP)
            ).reshape(npad, _EB)
    dst2d = jnp.concatenate([dst.reshape(_NBATCH, _EB), dump])
    idx_packed = jnp.stack([gidx2d, dst2d], axis=1)
    zer = jnp.zeros((_N, _D), jnp.float32)

    ft1, self1 = _transform(feat, V1, comp1, Wself1, b1)
    agg1 = _sc_agg(ft1.reshape(_R * _N, _D), idx_packed, zer)
    h1, ft2, self2 = _transform_fused(agg1, self1, feat, V2, comp2, Wself2, b2)
    agg2 = _sc_agg(ft2.reshape(_R * _N, _D), idx_packed, zer)
    h2, ft3, self3 = _transform_fused(agg2, self2, h1, V3, comp3, Wself3, b3)
    agg3 = _sc_agg(ft3.reshape(_R * _N, _D), idx_packed, zer)
    return _epilogue(agg3, self3, h2)


# speedup vs baseline: 3.5965x; 1.0133x over previous
import functools

import jax
import jax.numpy as jnp
from jax import lax
from jax.experimental import pallas as pl
from jax.experimental.pallas import tpu as pltpu
from jax.experimental.pallas import tpu_sc as plsc

_N = 10000
_E = 320000
_D = 128
_R = 8
_NB = 4

_BN = 2000
_GRID = _N // _BN
_EB = 128
_NBATCH = _E // _EB
_NWORK = 32
_T = 80
_NBATCH_PAD = _T * _NWORK
_NDUMP = 64
_NP = _N + _NDUMP
_RPS = 624
_RTAIL = _N - 16 * _RPS


def _gidx_body(et_ref, src_ref, out_ref):
    out_ref[...] = et_ref[...] * _N + src_ref[...]


def _make_gidx(etype, src):
    et2 = etype.reshape(_NBATCH, _EB)
    s2 = src.reshape(_NBATCH, _EB)
    return pl.pallas_call(
        _gidx_body,
        out_shape=jax.ShapeDtypeStruct((_NBATCH, _EB), jnp.int32),
    )(et2, s2)


def _transform_body(comp_ref, h_ref, V_ref, Wself_ref, b_ref, ft_ref, self_ref):
    h = h_ref[...]
    hb = [jnp.dot(h, V_ref[bb], preferred_element_type=jnp.float32)
          for bb in range(_NB)]
    for r in range(_R):
        acc = comp_ref[r, 0] * hb[0]
        for bb in range(1, _NB):
            acc = acc + comp_ref[r, bb] * hb[bb]
        ft_ref[r] = acc
    self_ref[...] = (jnp.dot(h, Wself_ref[...], preferred_element_type=jnp.float32)
                     + b_ref[...])


def _transform(h, V, comp, Wself, b):
    return pl.pallas_call(
        _transform_body,
        grid=(_GRID,),
        in_specs=[
            pl.BlockSpec(memory_space=pltpu.SMEM),
            pl.BlockSpec((_BN, _D), lambda i: (i, 0)),
            pl.BlockSpec((_NB, _D, _D), lambda i: (0, 0, 0)),
            pl.BlockSpec((_D, _D), lambda i: (0, 0)),
            pl.BlockSpec((1, _D), lambda i: (0, 0)),
        ],
        out_specs=[
            pl.BlockSpec((_R, _BN, _D), lambda i: (0, i, 0)),
            pl.BlockSpec((_BN, _D), lambda i: (i, 0)),
        ],
        out_shape=[
            jax.ShapeDtypeStruct((_R, _N, _D), jnp.float32),
            jax.ShapeDtypeStruct((_N, _D), jnp.float32),
        ],
    )(comp, h, V, Wself, b.reshape(1, _D))


def _transform_fused_body(comp_ref, agg_ref, selfp_ref, hprev_ref, V_ref,
                          Wself_ref, b_ref, h_ref, ft_ref, self_ref):
    h = (jnp.maximum(agg_ref[0] + agg_ref[1] + selfp_ref[...], 0.0)
         + hprev_ref[...])
    h_ref[...] = h
    hb = [jnp.dot(h, V_ref[bb], preferred_element_type=jnp.float32)
          for bb in range(_NB)]
    for r in range(_R):
        acc = comp_ref[r, 0] * hb[0]
        for bb in range(1, _NB):
            acc = acc + comp_ref[r, bb] * hb[bb]
        ft_ref[r] = acc
    self_ref[...] = (jnp.dot(h, Wself_ref[...], preferred_element_type=jnp.float32)
                     + b_ref[...])


def _transform_fused(agg, selfp, hprev, V, comp, Wself, b):
    return pl.pallas_call(
        _transform_fused_body,
        grid=(_GRID,),
        in_specs=[
            pl.BlockSpec(memory_space=pltpu.SMEM),
            pl.BlockSpec((2, _BN, _D), lambda i: (0, i, 0)),
            pl.BlockSpec((_BN, _D), lambda i: (i, 0)),
            pl.BlockSpec((_BN, _D), lambda i: (i, 0)),
            pl.BlockSpec((_NB, _D, _D), lambda i: (0, 0, 0)),
            pl.BlockSpec((_D, _D), lambda i: (0, 0)),
            pl.BlockSpec((1, _D), lambda i: (0, 0)),
        ],
        out_specs=[
            pl.BlockSpec((_BN, _D), lambda i: (i, 0)),
            pl.BlockSpec((_R, _BN, _D), lambda i: (0, i, 0)),
            pl.BlockSpec((_BN, _D), lambda i: (i, 0)),
        ],
        out_shape=[
            jax.ShapeDtypeStruct((_N, _D), jnp.float32),
            jax.ShapeDtypeStruct((_R, _N, _D), jnp.float32),
            jax.ShapeDtypeStruct((_N, _D), jnp.float32),
        ],
    )(comp, agg, selfp, hprev, V, Wself, b.reshape(1, _D))


def _epilogue_body(agg_ref, selfp_ref, hprev_ref, out_ref):
    out_ref[...] = (jnp.maximum(agg_ref[0] + agg_ref[1] + selfp_ref[...], 0.0)
                    + hprev_ref[...])


def _epilogue(agg, selfp, hprev):
    return pl.pallas_call(
        _epilogue_body,
        grid=(_GRID,),
        in_specs=[
            pl.BlockSpec((2, _BN, _D), lambda i: (0, i, 0)),
            pl.BlockSpec((_BN, _D), lambda i: (i, 0)),
            pl.BlockSpec((_BN, _D), lambda i: (i, 0)),
        ],
        out_specs=pl.BlockSpec((_BN, _D), lambda i: (i, 0)),
        out_shape=jax.ShapeDtypeStruct((_N, _D), jnp.float32),
    )(agg, selfp, hprev)


def _sc_agg_body(table_h, idx_h, zer_h, out_h,
                 pck0, pck1, pck2, rows0, rows1, rows2,
                 acc_s, semg0, semg1, semg2, sems0, sems1, sems2, semi):
    cid = lax.axis_index("c")
    sid = lax.axis_index("s")
    wid = sid * 2 + cid

    r0 = sid * _RPS
    pltpu.sync_copy(zer_h.at[pl.ds(r0, _RPS)], acc_s.at[pl.ds(r0, _RPS)])

    @pl.when(sid == 15)
    def _zero_tail():
        pltpu.sync_copy(zer_h.at[pl.ds(0, _RTAIL)],
                        acc_s.at[pl.ds(16 * _RPS, _RTAIL)])

    plsc.subcore_barrier()

    pck = (pck0, pck1, pck2)
    rows = (rows0, rows1, rows2)
    semg = (semg0, semg1, semg2)
    sems = (sems0, sems1, sems2)

    pltpu.sync_copy(idx_h.at[wid], pck[0])
    gdesc = [pltpu.async_copy(table_h.at[pck[0].at[0]], rows[0], semg[0]),
             None, None]
    sdesc = [None, None, None]
    idxd = pltpu.async_copy(idx_h.at[wid + _NWORK], pck[1], semi)

    for i in range(_T):
        k = i % 3
        kn = (i + 1) % 3
        if i + 1 < _T:
            idxd.wait()
            gdesc[kn] = pltpu.async_copy(table_h.at[pck[kn].at[0]],
                                         rows[kn], semg[kn])
        gdesc[k].wait()
        if i >= 1:
            sdesc[(i - 1) % 3].wait()
        sdesc[k] = pltpu.async_copy(rows[k], acc_s.at[pck[k].at[1]],
                                    sems[k], add=True)
        if i + 2 < _T:
            idxd = pltpu.async_copy(idx_h.at[wid + (i + 2) * _NWORK],
                                    pck[(i + 2) % 3], semi)

    sdesc[(_T - 1) % 3].wait()
    plsc.subcore_barrier()

    pltpu.sync_copy(acc_s.at[pl.ds(r0, _RPS)], out_h.at[cid, pl.ds(r0, _RPS)])

    @pl.when(sid == 15)
    def _write_tail():
        pltpu.sync_copy(acc_s.at[pl.ds(16 * _RPS, _RTAIL)],
                        out_h.at[cid, pl.ds(16 * _RPS, _RTAIL)])


@functools.cache
def _sc_agg_kernel():
    mesh = plsc.VectorSubcoreMesh(
        core_axis_name="c", subcore_axis_name="s", num_cores=2, num_subcores=16)
    return pl.kernel(
        _sc_agg_body,
        out_type=jax.ShapeDtypeStruct((2, _N, _D), jnp.float32),
        mesh=mesh,
        scratch_types=[
            pltpu.VMEM((2, _EB), jnp.int32),
            pltpu.VMEM((2, _EB), jnp.int32),
            pltpu.VMEM((2, _EB), jnp.int32),
            pltpu.VMEM((_EB, _D), jnp.float32),
            pltpu.VMEM((_EB, _D), jnp.float32),
            pltpu.VMEM((_EB, _D), jnp.float32),
            pltpu.VMEM_SHARED((_NP, _D), jnp.float32),
            pltpu.SemaphoreType.DMA,
            pltpu.SemaphoreType.DMA,
            pltpu.SemaphoreType.DMA,
            pltpu.SemaphoreType.DMA,
            pltpu.SemaphoreType.DMA,
            pltpu.SemaphoreType.DMA,
            pltpu.SemaphoreType.DMA,
        ],
    )


def _sc_agg(table, idx_packed, zer):
    return _sc_agg_kernel()(table, idx_packed, zer)


def kernel(feat, g, etype, V1, comp1, Wself1, b1, V2, comp2, Wself2, b2,
           V3, comp3, Wself3, b3):
    src = g[0]
    dst = g[1]
    npad = _NBATCH_PAD - _NBATCH
    padg = (jnp.arange(npad * _EB, dtype=jnp.int32) * 61 % (_R * _N)
            ).reshape(npad, _EB)
    gidx2d = jnp.concatenate([_make_gidx(etype, src), padg])
    dump = (_N + (jnp.arange(npad * _EB, dtype=jnp.int32) % _NDUMP)
            ).reshape(npad, _EB)
    dst2d = jnp.concatenate([dst.reshape(_NBATCH, _EB), dump])
    idx_packed = jnp.stack([gidx2d, dst2d], axis=1)
    zer = jnp.zeros((_N, _D), jnp.float32)

    ft1, self1 = _transform(feat, V1, comp1, Wself1, b1)
    agg1 = _sc_agg(ft1.reshape(_R * _N, _D), idx_packed, zer)
    h1, ft2, self2 = _transform_fused(agg1, self1, feat, V2, comp2, Wself2, b2)
    agg2 = _sc_agg(ft2.reshape(_R * _N, _D), idx_packed, zer)
    h2, ft3, self3 = _transform_fused(agg2, self2, h1, V3, comp3, Wself3, b3)
    agg3 = _sc_agg(ft3.reshape(_R * _N, _D), idx_packed, zer)
    return _epilogue(agg3, self3, h2)
